# baseline (speedup 1.0000x reference)
import jax
import jax.numpy as jnp
from jax import lax
from jax.experimental import pallas as pl
from jax.experimental.pallas import tpu as pltpu
from jax.experimental.pallas import tpu_sc as plsc

MAX_OBJ = 100
IOU_THRES = 0.45
SCORE_THRES = 0.25
NC = 80
MAX_WH = 640.0
N_BOXES = 20000
NPAD = 20480
NROW = NPAD // 128
NEG = -1.0e30
BIGI = 1 << 30

NT = 16
NB = NPAD // NT
CH = NB // 16


def _prep_kernel(xt_ref, f_ref):
    conf = xt_ref[4]
    cls = xt_ref[5:85]
    scores = conf[None] * cls
    cs = jnp.max(scores, axis=0)
    csub = lax.broadcasted_iota(jnp.int32, (NC, NROW, 128), 0).astype(
        jnp.float32)
    ci = (NC - 1) - jnp.max(
        jnp.where(scores == cs[None], (NC - 1) - csub, -1.0), axis=0)
    bx1 = xt_ref[0] - xt_ref[2] * 0.5
    by1 = xt_ref[1] - xt_ref[3] * 0.5
    bx2 = xt_ref[0] + xt_ref[2] * 0.5
    by2 = xt_ref[1] + xt_ref[3] * 0.5
    off = ci * MAX_WH
    ox1 = bx1 + off
    oy1 = by1 + off
    ox2 = bx2 + off
    oy2 = by2 + off
    f_ref[0] = jnp.where(cs > SCORE_THRES, cs, NEG)
    f_ref[1] = ox1
    f_ref[2] = oy1
    f_ref[3] = ox2
    f_ref[4] = oy2
    f_ref[5] = bx1
    f_ref[6] = by1
    f_ref[7] = bx2
    f_ref[8] = by2
    f_ref[9] = ci
    f_ref[10] = (ox2 - ox1) * (oy2 - oy1)
    f_ref[11] = jnp.zeros((NROW, 128), jnp.float32)


def _prep(xt):
    return pl.pallas_call(
        _prep_kernel,
        out_shape=jax.ShapeDtypeStruct((12, NROW, 128), jnp.float32),
    )(xt)


def _sc_nms_body(f_hbm, fb_hbm, out_hbm, fv, fvb, stage, allc, detb, shared):
    cid = lax.axis_index("c")
    tid = lax.axis_index("s")
    lane = lax.iota(jnp.int32, 16)

    @pl.when(cid == 0)
    def _():
        base = tid * NB
        for r in range(12):
            pltpu.sync_copy(f_hbm.at[r, pl.ds(base, NB)], fv.at[r])
        pltpu.sync_copy(fb_hbm.at[pl.ds(base * 16, NB * 16)], fvb)

        def argmax_of(vm, vi):
            m = vm[0]
            i = vi[0]
            for l in range(1, 16):
                vl = vm[l]
                il = vi[l]
                gt = vl > m
                eq = vl == m
                i = jnp.where(gt, il,
                              jnp.where(eq, jnp.minimum(i, il), i))
                m = jnp.where(gt, vl, m)
            return m, i

        def cand_of(m_t, i_t):
            c = fvb[pl.ds(pl.multiple_of(i_t * 16, 16), 16)]
            return jnp.where(lane == 0, m_t, c)

        def amax_body(c, carry):
            vm, vi = carry
            av = fv[0, pl.ds(pl.multiple_of(c * 16, 16), 16)]
            gt = av > vm
            vm = jnp.where(gt, av, vm)
            vi = jnp.where(gt, c * 16 + lane, vi)
            return vm, vi

        vm0 = jnp.full((16,), NEG, jnp.float32)
        vi0 = jnp.zeros((16,), jnp.int32)
        vm, vi = lax.fori_loop(0, CH, amax_body, (vm0, vi0))
        m_t, i_t = argmax_of(vm, vi)

        def body(it, carry):
            cand, i_sel = carry
            stage[...] = cand
            pltpu.sync_copy(stage, shared.at[tid])
            plsc.subcore_barrier()
            pltpu.sync_copy(shared, allc)
            plsc.subcore_barrier()

            def scan_body(t, c2):
                mb, wt = c2
                st = allc[t, pl.ds(0, 16)][0]
                better = st > mb
                return (jnp.where(better, st, mb),
                        jnp.where(better, t, wt))

            m, wt = lax.fori_loop(0, NT, scan_body,
                                  (jnp.float32(NEG), jnp.int32(0)))
            wrow = allc[wt, :]
            ok = m > 0.0
            wx1 = wrow[1]
            wy1 = wrow[2]
            wx2 = wrow[3]
            wy2 = wrow[4]
            wa = wrow[10]

            row = jnp.zeros((16,), jnp.float32)
            row = jnp.where(lane == 0, jnp.where(ok, wrow[5], 0.0), row)
            row = jnp.where(lane == 1, jnp.where(ok, wrow[6], 0.0), row)
            row = jnp.where(lane == 2, jnp.where(ok, wrow[7], 0.0), row)
            row = jnp.where(lane == 3, jnp.where(ok, wrow[8], 0.0), row)
            row = jnp.where(lane == 4, jnp.where(ok, wrow[0], 0.0), row)
            row = jnp.where(lane == 5, jnp.where(ok, wrow[9], -1.0), row)
            detb[it, :] = row

            thr = jnp.where(ok, IOU_THRES, 3.0e38)
            isel_eff = jnp.where(wt == tid, i_sel, -1)

            def fuse_body(c, c3):
                vm, vi = c3
                sl = pl.ds(pl.multiple_of(c * 16, 16), 16)
                av = fv[0, sl]
                ix1 = jnp.maximum(wx1, fv[1, sl])
                iy1 = jnp.maximum(wy1, fv[2, sl])
                ix2 = jnp.minimum(wx2, fv[3, sl])
                iy2 = jnp.minimum(wy2, fv[4, sl])
                inter = (jnp.maximum(ix2 - ix1, 0.0)
                         * jnp.maximum(iy2 - iy1, 0.0))
                iou = inter / (wa + fv[10, sl] - inter + 1e-9)
                gidx = c * 16 + lane
                nav = jnp.where(iou > thr, NEG,
                                jnp.where(gidx == isel_eff, NEG, av))
                fv[0, sl] = nav
                gt = nav > vm
                vm = jnp.where(gt, nav, vm)
                vi = jnp.where(gt, gidx, vi)
                return vm, vi

            vm, vi = lax.fori_loop(0, CH, fuse_body, (vm0, vi0))
            m_n, i_n = argmax_of(vm, vi)
            return cand_of(m_n, i_n), i_n

        lax.fori_loop(0, MAX_OBJ, body, (cand_of(m_t, i_t), i_t))

        @pl.when(tid == 0)
        def _():
            pltpu.sync_copy(detb, out_hbm)


def _sc_nms(fields, fbox):
    mesh = plsc.VectorSubcoreMesh(
        core_axis_name="c", subcore_axis_name="s", num_cores=1)
    return pl.kernel(
        _sc_nms_body,
        out_type=jax.ShapeDtypeStruct((MAX_OBJ, 16), jnp.float32),
        mesh=mesh,
        scratch_types=[
            pltpu.VMEM((12, NB), jnp.float32),
            pltpu.VMEM((NB * 16,), jnp.float32),
            pltpu.VMEM((16,), jnp.float32),
            pltpu.VMEM((16, 16), jnp.float32),
            pltpu.VMEM((MAX_OBJ, 16), jnp.float32),
            pltpu.VMEM_SHARED((16, 16), jnp.float32),
        ],
    )(fields, fbox)


def kernel(x):
    xp = jnp.pad(x[0], ((0, NPAD - N_BOXES), (0, 0)))
    xt = xp.T.reshape(85, NROW, 128)
    fields = _prep(xt).reshape(12, NPAD)
    fbox = jnp.pad(fields.T, ((0, 0), (0, 4))).reshape(-1)
    dets = _sc_nms(fields, fbox)
    return dets[None, :, :6]
